# Initial kernel scaffold; baseline (speedup 1.0000x reference)
import jax
import jax.numpy as jnp
from jax.experimental import pallas as pl


def kernel(input_ids, word_emb, pos_emb, type_emb, ln_gamma, ln_beta):
    raise NotImplementedError("write your pallas kernel here")


# speedup vs baseline: 1.8096x; 1.8096x over previous
import functools

import jax
import jax.numpy as jnp
from jax import lax
from jax.experimental import pallas as pl
from jax.experimental.pallas import tpu as pltpu
from jax.experimental.pallas import tpu_sc as plsc

VOCAB = 50265
HID = 768
MAXPOS = 2050
TYPEV = 2
PAD = 1
EPS = 1e-12
B = 4
S = 2048
N = B * S

_NC = 2
_NS = 16
_NW = _NC * _NS
_B_PER_W = N // _NW
_CH = 64


def _posid_body(ids_ref, out_ref):
    ids = ids_ref[...]
    mask = (ids != PAD)
    maskb = mask.astype(jnp.bfloat16)
    ri = lax.broadcasted_iota(jnp.int32, (S, S), 0)
    ci = lax.broadcasted_iota(jnp.int32, (S, S), 1)
    tri = (ri <= ci).astype(jnp.bfloat16)
    inc = lax.dot_general(maskb, tri, (((1,), (0,)), ((), ())),
                          preferred_element_type=jnp.float32)
    out_ref[...] = inc.astype(jnp.int32) * mask.astype(jnp.int32) + PAD


_posid_call = pl.pallas_call(
    _posid_body,
    out_shape=jax.ShapeDtypeStruct((B, S), jnp.int32),
)


def _gather_body(word_hbm, pos_hbm, wid_hbm, pid_hbm, ow_hbm, op_hbm,
                 idw_v, idp_v, rw_v, rp_v, semw, semp):
    w = lax.axis_index("s") * _NC + lax.axis_index("c")
    base = w * _B_PER_W

    @pl.loop(0, _B_PER_W, step=_CH)
    def _(c):
        off = base + c
        pltpu.sync_copy(wid_hbm.at[pl.ds(off, _CH)], idw_v)
        pltpu.sync_copy(pid_hbm.at[pl.ds(off, _CH)], idp_v)
        cpw = pltpu.async_copy(word_hbm.at[idw_v], rw_v, semw)
        cpp = pltpu.async_copy(pos_hbm.at[idp_v], rp_v, semp)
        cpw.wait()
        cpp.wait()
        pltpu.sync_copy(rw_v, ow_hbm.at[pl.ds(off, _CH)])
        pltpu.sync_copy(rp_v, op_hbm.at[pl.ds(off, _CH)])


@functools.cache
def _gather_call():
    return functools.partial(
        pl.kernel,
        out_type=(jax.ShapeDtypeStruct((N, HID), jnp.float32),
                  jax.ShapeDtypeStruct((N, HID), jnp.float32)),
        mesh=plsc.VectorSubcoreMesh(core_axis_name="c", subcore_axis_name="s"),
        scratch_types=[
            pltpu.VMEM((_CH,), jnp.int32),
            pltpu.VMEM((_CH,), jnp.int32),
            pltpu.VMEM((_CH, HID), jnp.float32),
            pltpu.VMEM((_CH, HID), jnp.float32),
            pltpu.SemaphoreType.DMA,
            pltpu.SemaphoreType.DMA,
        ],
    )(_gather_body)


_LN_BLK = 1024


def _ln_body(gw_ref, gp_ref, type_ref, g_ref, b_ref, o_ref):
    x = gw_ref[...] + gp_ref[...] + type_ref[0, :][None, :]
    mean = jnp.mean(x, axis=-1, keepdims=True)
    xc = x - mean
    var = jnp.mean(xc * xc, axis=-1, keepdims=True)
    o_ref[...] = (xc * lax.rsqrt(var + EPS)) * g_ref[0, :][None, :] \
        + b_ref[0, :][None, :]


_ln_call = pl.pallas_call(
    _ln_body,
    grid=(N // _LN_BLK,),
    in_specs=[
        pl.BlockSpec((_LN_BLK, HID), lambda i: (i, 0)),
        pl.BlockSpec((_LN_BLK, HID), lambda i: (i, 0)),
        pl.BlockSpec((TYPEV, HID), lambda i: (0, 0)),
        pl.BlockSpec((1, HID), lambda i: (0, 0)),
        pl.BlockSpec((1, HID), lambda i: (0, 0)),
    ],
    out_specs=pl.BlockSpec((_LN_BLK, HID), lambda i: (i, 0)),
    out_shape=jax.ShapeDtypeStruct((N, HID), jnp.float32),
)


def kernel(input_ids, word_emb, pos_emb, type_emb, ln_gamma, ln_beta):
    position_ids = _posid_call(input_ids)
    gw, gp = _gather_call()(word_emb, pos_emb,
                          input_ids.reshape(N), position_ids.reshape(N))
    out = _ln_call(gw, gp, type_emb,
                   ln_gamma.reshape(1, HID), ln_beta.reshape(1, HID))
    return out.reshape(B, S, HID), position_ids
